# baseline (speedup 1.0000x reference)
import functools

import jax
import jax.numpy as jnp
from jax import lax
from jax.experimental import pallas as pl
from jax.experimental.pallas import tpu as pltpu
from jax.experimental.pallas import tpu_sc as plsc

N_NODES = 10000
D = 128
HD = D // 2
E = 320000
NC = 2
NS = 16
GE = 160
NG = 128
EPW = NG * GE
EPW_REAL = E // NS
NPAD = EPW - EPW_REAL
ACC_PAD = 16
ACC_ROWS = N_NODES + ACC_PAD
SPAN = 624

_mesh = plsc.VectorSubcoreMesh(core_axis_name="c", subcore_axis_name="s")


@functools.partial(
    pl.kernel,
    mesh=_mesh,
    out_type=jax.ShapeDtypeStruct((NC, N_NODES, HD), jnp.float32),
    scratch_types=[
        pltpu.VMEM((EPW,), jnp.int32),
        pltpu.VMEM((EPW,), jnp.int32),
        pltpu.VMEM((GE, HD), jnp.float32),
        pltpu.VMEM((GE, HD), jnp.float32),
        pltpu.VMEM((GE, HD), jnp.float32),
        pltpu.VMEM((GE, HD), jnp.float32),
        pltpu.VMEM_SHARED((ACC_ROWS, HD), jnp.float32),
        pltpu.SemaphoreType.DMA,
        pltpu.SemaphoreType.DMA,
        pltpu.SemaphoreType.DMA,
        pltpu.SemaphoreType.DMA,
        pltpu.SemaphoreType.DMA,
        pltpu.SemaphoreType.DMA,
        pltpu.SemaphoreType.DMA,
        pltpu.SemaphoreType.DMA,
    ],
    compiler_params=pltpu.CompilerParams(use_tc_tiling_on_sc=False),
)
def _sc_aggregate(edge_hbm, feat_hbm, out_hbm,
                  src_v, dst_v, buf_0, buf_1, buf_2, buf_3, acc,
                  gs_0, gs_1, gs_2, gs_3, ss_0, ss_1, ss_2, ss_3):
    buf_a = buf_0
    bufs = (buf_0, buf_1, buf_2, buf_3)
    gsems = (gs_0, gs_1, gs_2, gs_3)
    ssems = (ss_0, ss_1, ss_2, ss_3)
    c = lax.axis_index("c")
    s = lax.axis_index("s")

    zero16 = jnp.zeros((16,), jnp.float32)

    def _zbody(i, carry):
        buf_a[i // (HD // 16), pl.ds((i % (HD // 16)) * 16, 16)] = zero16
        return carry

    lax.fori_loop(0, (128 * HD) // 16, _zbody, 0)

    zpage = buf_a.at[pl.ds(0, 128)]
    zbase = s * SPAN
    pltpu.sync_copy(zpage, acc.at[pl.ds(zbase, 128)])
    pltpu.sync_copy(zpage, acc.at[pl.ds(zbase + 128, 128)])
    pltpu.sync_copy(zpage, acc.at[pl.ds(zbase + 2 * 128, 128)])
    pltpu.sync_copy(zpage, acc.at[pl.ds(zbase + 3 * 128, 128)])
    pltpu.sync_copy(zpage.at[pl.ds(0, SPAN - 4 * 128)],
                    acc.at[pl.ds(zbase + 4 * 128, SPAN - 4 * 128)])

    @pl.when(s == NS - 1)
    def _ztail():
        pltpu.sync_copy(zpage.at[pl.ds(0, ACC_ROWS - NS * SPAN)],
                        acc.at[pl.ds(NS * SPAN, ACC_ROWS - NS * SPAN)])

    plsc.subcore_barrier()

    ebase = s * EPW_REAL
    pltpu.sync_copy(edge_hbm.at[0].at[pl.ds(ebase, EPW_REAL)],
                    src_v.at[pl.ds(0, EPW_REAL)])
    pltpu.sync_copy(edge_hbm.at[1].at[pl.ds(ebase, EPW_REAL)],
                    dst_v.at[pl.ds(0, EPW_REAL)])

    iota16 = lax.iota(jnp.int32, 16)
    src_pad = iota16 + s * 97
    dst_pad = iota16 + jnp.int32(N_NODES)

    def _pbody(i, carry):
        src_v[pl.ds(EPW_REAL + i * 16, 16)] = src_pad
        dst_v[pl.ds(EPW_REAL + i * 16, 16)] = dst_pad
        return carry

    lax.fori_loop(0, NPAD // 16, _pbody, 0)

    feat_c = feat_hbm.at[c]

    def _sidx(g):
        return src_v.at[pl.ds(g * GE, GE)]

    def _didx(g):
        return dst_v.at[pl.ds(g * GE, GE)]

    pltpu.async_copy(feat_c.at[_sidx(0)], bufs[0], gsems[0])
    pltpu.async_copy(feat_c.at[_sidx(1)], bufs[1], gsems[1])

    NJ = NG // 4

    def _body(jj, carry):
        for i in range(4):
            g = 4 * jj + i
            b = (i + 2) % 4
            pltpu.make_async_copy(feat_c.at[_sidx(g)], bufs[i], gsems[i]).wait()
            pltpu.async_copy(bufs[i], acc.at[_didx(g)], ssems[i], add=True)

            if i < 2:
                @pl.when(jj == 0)
                def _refill_first(b=b, g=g):
                    pltpu.async_copy(feat_c.at[_sidx(g + 2)], bufs[b],
                                     gsems[b])

                @pl.when(jj >= 1)
                def _wait_refill(b=b, g=g):
                    pltpu.make_async_copy(bufs[b], acc.at[_didx(g)],
                                          ssems[b]).wait()
                    pltpu.async_copy(feat_c.at[_sidx(g + 2)], bufs[b],
                                     gsems[b])
            else:
                @pl.when(jj < NJ - 1)
                def _wait_refill(b=b, g=g):
                    pltpu.make_async_copy(bufs[b], acc.at[_didx(g)],
                                          ssems[b]).wait()
                    pltpu.async_copy(feat_c.at[_sidx(g + 2)], bufs[b],
                                     gsems[b])
        return carry

    lax.fori_loop(0, NJ, _body, 0)

    for i in range(4):
        pltpu.make_async_copy(bufs[i], acc.at[_didx(NG - 4 + i)],
                              ssems[i]).wait()

    plsc.subcore_barrier()

    obase = s * SPAN
    pltpu.sync_copy(acc.at[pl.ds(obase, SPAN)],
                    out_hbm.at[c].at[pl.ds(obase, SPAN)])

    @pl.when(s == NS - 1)
    def _otail():
        pltpu.sync_copy(acc.at[pl.ds(NS * SPAN, N_NODES - NS * SPAN)],
                        out_hbm.at[c].at[pl.ds(NS * SPAN, N_NODES - NS * SPAN)])


BLK = 1000


def _tc_body(p_ref, wt_ref, b_ref, o_ref):
    agg = jnp.concatenate([p_ref[0], p_ref[1]], axis=-1)
    o_ref[...] = (
        jnp.dot(agg, wt_ref[...], preferred_element_type=jnp.float32)
        + b_ref[...]
    )


def _tc_linear(partials, wt, b2):
    return pl.pallas_call(
        _tc_body,
        grid=(N_NODES // BLK,),
        in_specs=[
            pl.BlockSpec((NC, BLK, HD), lambda i: (0, i, 0)),
            pl.BlockSpec((D, D), lambda i: (0, 0)),
            pl.BlockSpec((1, D), lambda i: (0, 0)),
        ],
        out_specs=pl.BlockSpec((BLK, D), lambda i: (i, 0)),
        out_shape=jax.ShapeDtypeStruct((N_NODES, D), jnp.float32),
    )(partials, wt, b2)


def kernel(feat, edge_index, W, b):
    edges = edge_index.astype(jnp.int32)
    feat_halves = jnp.stack([feat[:, :HD], feat[:, HD:]])
    partials = _sc_aggregate(edges, feat_halves)
    return _tc_linear(partials, W.T, b.reshape(1, D))


# speedup vs baseline: 1.0586x; 1.0586x over previous
import functools

import jax
import jax.numpy as jnp
from jax import lax
from jax.experimental import pallas as pl
from jax.experimental.pallas import tpu as pltpu
from jax.experimental.pallas import tpu_sc as plsc

N_NODES = 10000
D = 128
HD = D // 2
E = 320000
NC = 2
NS = 16
GE = 320
NG = 64
EPW = NG * GE
EPW_REAL = E // NS
NPAD = EPW - EPW_REAL
ACC_PAD = 16
ACC_ROWS = N_NODES + ACC_PAD
SPAN = 624

_mesh = plsc.VectorSubcoreMesh(core_axis_name="c", subcore_axis_name="s")


@functools.partial(
    pl.kernel,
    mesh=_mesh,
    out_type=jax.ShapeDtypeStruct((NC, N_NODES, HD), jnp.float32),
    scratch_types=[
        pltpu.VMEM((EPW,), jnp.int32),
        pltpu.VMEM((EPW,), jnp.int32),
        pltpu.VMEM((GE, HD), jnp.float32),
        pltpu.VMEM((GE, HD), jnp.float32),
        pltpu.VMEM_SHARED((ACC_ROWS, HD), jnp.float32),
        pltpu.SemaphoreType.DMA,
        pltpu.SemaphoreType.DMA,
    ],
    compiler_params=pltpu.CompilerParams(use_tc_tiling_on_sc=False),
)
def _sc_aggregate(edge_hbm, feat_hbm, out_hbm,
                  src_v, dst_v, buf_a, buf_b, acc, sem_a, sem_b):
    c = lax.axis_index("c")
    s = lax.axis_index("s")

    zero16 = jnp.zeros((16,), jnp.float32)

    def _zbody(i, carry):
        buf_a[i // (HD // 16), pl.ds((i % (HD // 16)) * 16, 16)] = zero16
        return carry

    lax.fori_loop(0, (128 * HD) // 16, _zbody, 0)

    zpage = buf_a.at[pl.ds(0, 128)]
    zbase = s * SPAN
    pltpu.sync_copy(zpage, acc.at[pl.ds(zbase, 128)])
    pltpu.sync_copy(zpage, acc.at[pl.ds(zbase + 128, 128)])
    pltpu.sync_copy(zpage, acc.at[pl.ds(zbase + 2 * 128, 128)])
    pltpu.sync_copy(zpage, acc.at[pl.ds(zbase + 3 * 128, 128)])
    pltpu.sync_copy(zpage.at[pl.ds(0, SPAN - 4 * 128)],
                    acc.at[pl.ds(zbase + 4 * 128, SPAN - 4 * 128)])

    @pl.when(s == NS - 1)
    def _ztail():
        pltpu.sync_copy(zpage.at[pl.ds(0, ACC_ROWS - NS * SPAN)],
                        acc.at[pl.ds(NS * SPAN, ACC_ROWS - NS * SPAN)])

    plsc.subcore_barrier()

    ebase = s * EPW_REAL
    pltpu.sync_copy(edge_hbm.at[0].at[pl.ds(ebase, EPW_REAL)],
                    src_v.at[pl.ds(0, EPW_REAL)])
    pltpu.sync_copy(edge_hbm.at[1].at[pl.ds(ebase, EPW_REAL)],
                    dst_v.at[pl.ds(0, EPW_REAL)])

    iota16 = lax.iota(jnp.int32, 16)
    src_pad = iota16 + s * 97
    dst_pad = iota16 + jnp.int32(N_NODES)

    def _pbody(i, carry):
        src_v[pl.ds(EPW_REAL + i * 16, 16)] = src_pad
        dst_v[pl.ds(EPW_REAL + i * 16, 16)] = dst_pad
        return carry

    lax.fori_loop(0, NPAD // 16, _pbody, 0)

    feat_c = feat_hbm.at[c]

    def _sidx(g):
        return src_v.at[pl.ds(g * GE, GE)]

    def _didx(g):
        return dst_v.at[pl.ds(g * GE, GE)]

    pltpu.async_copy(feat_c.at[_sidx(0)], buf_a, sem_a)

    def _body(jj, carry):
        g0 = 2 * jj
        pltpu.async_copy(feat_c.at[_sidx(g0 + 1)], buf_b, sem_b)
        pltpu.make_async_copy(feat_c.at[_sidx(g0)], buf_a, sem_a).wait()
        pltpu.sync_copy(buf_a, acc.at[_didx(g0)], add=True)

        @pl.when(jj < NG // 2 - 1)
        def _():
            pltpu.async_copy(feat_c.at[_sidx(g0 + 2)], buf_a, sem_a)

        pltpu.make_async_copy(feat_c.at[_sidx(g0 + 1)], buf_b, sem_b).wait()
        pltpu.sync_copy(buf_b, acc.at[_didx(g0 + 1)], add=True)
        return carry

    lax.fori_loop(0, NG // 2, _body, 0)

    plsc.subcore_barrier()

    obase = s * SPAN
    pltpu.sync_copy(acc.at[pl.ds(obase, SPAN)],
                    out_hbm.at[c].at[pl.ds(obase, SPAN)])

    @pl.when(s == NS - 1)
    def _otail():
        pltpu.sync_copy(acc.at[pl.ds(NS * SPAN, N_NODES - NS * SPAN)],
                        out_hbm.at[c].at[pl.ds(NS * SPAN, N_NODES - NS * SPAN)])


BLK = 1000


def _tc_body(p_ref, wt_ref, b_ref, o_ref):
    agg = jnp.concatenate([p_ref[0], p_ref[1]], axis=-1)
    o_ref[...] = (
        jnp.dot(agg, wt_ref[...], preferred_element_type=jnp.float32)
        + b_ref[...]
    )


def _tc_linear(partials, wt, b2):
    return pl.pallas_call(
        _tc_body,
        grid=(N_NODES // BLK,),
        in_specs=[
            pl.BlockSpec((NC, BLK, HD), lambda i: (0, i, 0)),
            pl.BlockSpec((D, D), lambda i: (0, 0)),
            pl.BlockSpec((1, D), lambda i: (0, 0)),
        ],
        out_specs=pl.BlockSpec((BLK, D), lambda i: (i, 0)),
        out_shape=jax.ShapeDtypeStruct((N_NODES, D), jnp.float32),
        compiler_params=pltpu.CompilerParams(
            allow_input_fusion=[True, False, False]),
    )(partials, wt, b2)


def kernel(feat, edge_index, W, b):
    edges = edge_index.astype(jnp.int32)
    feat_halves = jnp.stack([feat[:, :HD], feat[:, HD:]])
    partials = _sc_aggregate(edges, feat_halves)
    return _tc_linear(partials, W.T, b.reshape(1, D))


# speedup vs baseline: 1.0776x; 1.0179x over previous
import functools

import jax
import jax.numpy as jnp
from jax import lax
from jax.experimental import pallas as pl
from jax.experimental.pallas import tpu as pltpu
from jax.experimental.pallas import tpu_sc as plsc

N_NODES = 10000
D = 128
HD = D // 2
E = 320000
NC = 2
NS = 16
GE = 320
NG = 64
EPW = NG * GE
EPW_REAL = E // NS
NPAD = EPW - EPW_REAL
ACC_PAD = 16
ACC_ROWS = N_NODES + ACC_PAD
SPAN = 624

_mesh = plsc.VectorSubcoreMesh(core_axis_name="c", subcore_axis_name="s")


@functools.partial(
    pl.kernel,
    mesh=_mesh,
    out_type=jax.ShapeDtypeStruct((NC, N_NODES, HD), jnp.float32),
    scratch_types=[
        pltpu.VMEM((EPW,), jnp.int32),
        pltpu.VMEM((EPW,), jnp.int32),
        pltpu.VMEM((GE, HD), jnp.float32),
        pltpu.VMEM((GE, HD), jnp.float32),
        pltpu.VMEM_SHARED((ACC_ROWS, HD), jnp.float32),
        pltpu.SemaphoreType.DMA,
        pltpu.SemaphoreType.DMA,
    ],
    compiler_params=pltpu.CompilerParams(use_tc_tiling_on_sc=False),
)
def _sc_aggregate(edge_hbm, feat_hbm, out_hbm,
                  src_v, dst_v, buf_a, buf_b, acc, sem_a, sem_b):
    c = lax.axis_index("c")
    s = lax.axis_index("s")

    ebase = s * EPW_REAL
    pltpu.async_copy(edge_hbm.at[0].at[pl.ds(ebase, EPW_REAL)],
                     src_v.at[pl.ds(0, EPW_REAL)], sem_a)
    pltpu.async_copy(edge_hbm.at[1].at[pl.ds(ebase, EPW_REAL)],
                     dst_v.at[pl.ds(0, EPW_REAL)], sem_b)

    zero16 = jnp.zeros((16,), jnp.float32)

    def _zbody(i, carry):
        buf_a[i // (HD // 16), pl.ds((i % (HD // 16)) * 16, 16)] = zero16
        return carry

    lax.fori_loop(0, (128 * HD) // 16, _zbody, 0)

    zpage = buf_a.at[pl.ds(0, 128)]
    zbase = s * SPAN
    pltpu.sync_copy(zpage, acc.at[pl.ds(zbase, 128)])
    pltpu.sync_copy(zpage, acc.at[pl.ds(zbase + 128, 128)])
    pltpu.sync_copy(zpage, acc.at[pl.ds(zbase + 2 * 128, 128)])
    pltpu.sync_copy(zpage, acc.at[pl.ds(zbase + 3 * 128, 128)])
    pltpu.sync_copy(zpage.at[pl.ds(0, SPAN - 4 * 128)],
                    acc.at[pl.ds(zbase + 4 * 128, SPAN - 4 * 128)])

    @pl.when(s == NS - 1)
    def _ztail():
        pltpu.sync_copy(zpage.at[pl.ds(0, ACC_ROWS - NS * SPAN)],
                        acc.at[pl.ds(NS * SPAN, ACC_ROWS - NS * SPAN)])

    plsc.subcore_barrier()

    pltpu.make_async_copy(edge_hbm.at[0].at[pl.ds(ebase, EPW_REAL)],
                          src_v.at[pl.ds(0, EPW_REAL)], sem_a).wait()
    pltpu.make_async_copy(edge_hbm.at[1].at[pl.ds(ebase, EPW_REAL)],
                          dst_v.at[pl.ds(0, EPW_REAL)], sem_b).wait()

    iota16 = lax.iota(jnp.int32, 16)
    src_pad = iota16 + s * 97
    dst_pad = iota16 + jnp.int32(N_NODES)

    def _pbody(i, carry):
        src_v[pl.ds(EPW_REAL + i * 16, 16)] = src_pad
        dst_v[pl.ds(EPW_REAL + i * 16, 16)] = dst_pad
        return carry

    lax.fori_loop(0, NPAD // 16, _pbody, 0)

    feat_c = feat_hbm.at[c]

    def _sidx(g):
        return src_v.at[pl.ds(g * GE, GE)]

    def _didx(g):
        return dst_v.at[pl.ds(g * GE, GE)]

    pltpu.async_copy(feat_c.at[_sidx(0)], buf_a, sem_a)

    def _body(jj, carry):
        g0 = 2 * jj
        pltpu.async_copy(feat_c.at[_sidx(g0 + 1)], buf_b, sem_b)
        pltpu.make_async_copy(feat_c.at[_sidx(g0)], buf_a, sem_a).wait()
        pltpu.sync_copy(buf_a, acc.at[_didx(g0)], add=True)

        @pl.when(jj < NG // 2 - 1)
        def _():
            pltpu.async_copy(feat_c.at[_sidx(g0 + 2)], buf_a, sem_a)

        pltpu.make_async_copy(feat_c.at[_sidx(g0 + 1)], buf_b, sem_b).wait()
        pltpu.sync_copy(buf_b, acc.at[_didx(g0 + 1)], add=True)
        return carry

    lax.fori_loop(0, NG // 2, _body, 0)

    plsc.subcore_barrier()

    obase = s * SPAN
    pltpu.sync_copy(acc.at[pl.ds(obase, SPAN)],
                    out_hbm.at[c].at[pl.ds(obase, SPAN)])

    @pl.when(s == NS - 1)
    def _otail():
        pltpu.sync_copy(acc.at[pl.ds(NS * SPAN, N_NODES - NS * SPAN)],
                        out_hbm.at[c].at[pl.ds(NS * SPAN, N_NODES - NS * SPAN)])


BLK = 1000


def _tc_body(p_ref, wt_ref, b_ref, o_ref):
    agg = jnp.concatenate([p_ref[0], p_ref[1]], axis=-1)
    o_ref[...] = (
        jnp.dot(agg, wt_ref[...], preferred_element_type=jnp.float32)
        + b_ref[...]
    )


def _tc_linear(partials, wt, b2):
    return pl.pallas_call(
        _tc_body,
        grid=(N_NODES // BLK,),
        in_specs=[
            pl.BlockSpec((NC, BLK, HD), lambda i: (0, i, 0)),
            pl.BlockSpec((D, D), lambda i: (0, 0)),
            pl.BlockSpec((1, D), lambda i: (0, 0)),
        ],
        out_specs=pl.BlockSpec((BLK, D), lambda i: (i, 0)),
        out_shape=jax.ShapeDtypeStruct((N_NODES, D), jnp.float32),
        compiler_params=pltpu.CompilerParams(
            allow_input_fusion=[True, False, False]),
    )(partials, wt, b2)


def kernel(feat, edge_index, W, b):
    edges = edge_index.astype(jnp.int32)
    feat_halves = jnp.stack([feat[:, :HD], feat[:, HD:]])
    partials = _sc_aggregate(edges, feat_halves)
    return _tc_linear(partials, W.T, b.reshape(1, D))
